# Initial kernel scaffold; baseline (speedup 1.0000x reference)
import jax
import jax.numpy as jnp
from jax.experimental import pallas as pl


def kernel(x, edge_index_0, edge_index_1, edge_index_2, edge_index_3, W1_0, W1_1, W1_2, W1_3, b1_0, b1_1, b1_2, b1_3, W2_0, W2_1, W2_2, W2_3, b2_0, b2_1, b2_2, b2_3):
    raise NotImplementedError("write your pallas kernel here")


# speedup vs baseline: 2.4301x; 2.4301x over previous
import functools

import jax
import jax.numpy as jnp
from jax import lax
from jax.experimental import pallas as pl
from jax.experimental.pallas import tpu as pltpu
from jax.experimental.pallas import tpu_sc as plsc

N = 10000
E = 80000
R = 4
D = 128

NC = 2
NS = 16
L = 16
W = NC * NS

K = 128
NCHUNK = E // K
NP = 10240
RPT = NP // NS
ZR = 128

BN = 1000


def _hist_body(e_s0, e_d0, e_s1, e_d1, e_s2, e_d2, e_s3, e_d3,
               out_hbm, hist, ibuf):
    cid = lax.axis_index("c")
    sid = lax.axis_index("s")
    w = cid * NS + sid

    zv = jnp.zeros((L,), jnp.float32)

    def zero_body(i, _):
        hist[pl.ds(i * L, L)] = zv
        return 0

    lax.fori_loop(0, N * 8 // L, zero_body, 0)

    ones = jnp.ones((L,), jnp.float32)
    streams = (e_s0, e_d0, e_s1, e_d1, e_s2, e_d2, e_s3, e_d3)
    for col, e in enumerate(streams):
        nmy = (NCHUNK - 1 - w) // W + 1

        def body(k, _, e=e, col=col):
            c = w + k * W
            pltpu.sync_copy(e.at[pl.ds(c * K, K)], ibuf)
            for i in range(K // L):
                idx = ibuf[pl.ds(i * L, L)]
                plsc.addupdate_scatter(hist, [idx * 8 + col], ones)
            return 0

        lax.fori_loop(0, nmy, body, 0)

    pltpu.sync_copy(hist, out_hbm.at[w])


_hist_call = pl.kernel(
    _hist_body,
    out_type=jax.ShapeDtypeStruct((W, N * 8), jnp.float32),
    mesh=plsc.VectorSubcoreMesh(core_axis_name="c", subcore_axis_name="s"),
    compiler_params=pltpu.CompilerParams(needs_layout_passes=False),
    scratch_types=[
        pltpu.VMEM((N * 8,), jnp.float32),
        pltpu.VMEM((K,), jnp.int32),
    ],
)


def _agg_body(y0, y1, y2, y3, e_s0, e_d0, e_s1, e_d1, e_s2, e_d2, e_s3, e_d3,
              out_hbm, acc, rows, sbuf, dbuf, zbuf, sem):
    cid = lax.axis_index("c")
    sid = lax.axis_index("s")
    w = cid * NS + sid

    zv = jnp.zeros((L,), jnp.float32)

    def zfill(i, _):
        zbuf[i // (D // L), pl.ds((i % (D // L)) * L, L)] = zv
        return 0

    lax.fori_loop(0, ZR * D // L, zfill, 0)

    ys = (y0, y1, y2, y3)
    srcs = (e_s0, e_s1, e_s2, e_s3)
    dsts = (e_d0, e_d1, e_d2, e_d3)
    for r in range(R):
        for j in range(RPT // ZR):
            pltpu.sync_copy(zbuf, acc.at[pl.ds(sid * RPT + j * ZR, ZR)])
        plsc.subcore_barrier()

        nmy = (NCHUNK - 1 - w) // W + 1

        def body(k, _, y=ys[r], es=srcs[r], ed=dsts[r]):
            off = (w + k * W) * K
            pltpu.sync_copy(es.at[pl.ds(off, K)], sbuf)
            pltpu.sync_copy(ed.at[pl.ds(off, K)], dbuf)
            pltpu.async_copy(y.at[sbuf], rows, sem).wait()
            pltpu.sync_copy(rows, acc.at[dbuf], add=True)
            return 0

        lax.fori_loop(0, nmy, body, 0)
        plsc.subcore_barrier()

        slot = cid * R + r
        for j in range(RPT // ZR):
            ro = sid * RPT + j * ZR
            pltpu.sync_copy(acc.at[pl.ds(ro, ZR)],
                            out_hbm.at[pl.ds(slot * NP + ro, ZR)])
        plsc.subcore_barrier()


_agg_call = pl.kernel(
    _agg_body,
    out_type=jax.ShapeDtypeStruct((NC * R * NP, D), jnp.float32),
    mesh=plsc.VectorSubcoreMesh(core_axis_name="c", subcore_axis_name="s"),
    compiler_params=pltpu.CompilerParams(needs_layout_passes=False),
    scratch_types=[
        pltpu.VMEM_SHARED((NP, D), jnp.float32),
        pltpu.VMEM((K, D), jnp.float32),
        pltpu.VMEM((K,), jnp.int32),
        pltpu.VMEM((K,), jnp.int32),
        pltpu.VMEM((ZR, D), jnp.float32),
        pltpu.SemaphoreType.DMA,
    ],
)


def _t1_body(x_ref, cnt_ref, w1_ref, y0_ref, y1_ref, y2_ref, y3_ref,
             rdeg_ref):
    cnt = jnp.sum(cnt_ref[...], axis=0)
    rdeg = lax.rsqrt(jnp.maximum(cnt, 1.0))
    rdeg_ref[...] = rdeg
    x = x_ref[...]
    outs = (y0_ref, y1_ref, y2_ref, y3_ref)
    for r in range(R):
        xs = x * rdeg[:, 2 * r:2 * r + 1]
        outs[r][...] = jnp.dot(xs, w1_ref[r],
                               preferred_element_type=jnp.float32)


def _t2_body(p_ref, rdeg_ref, b1_ref, w2_ref, y0_ref, y1_ref, y2_ref,
             y3_ref):
    p = p_ref[...]
    rdeg = rdeg_ref[...]
    h = jnp.sum(b1_ref[...], axis=0)[None, :]
    for r in range(R):
        h = h + rdeg[:, 2 * r + 1:2 * r + 2] * (p[0, r] + p[1, r])
    h = jnp.maximum(h, 0.0)
    outs = (y0_ref, y1_ref, y2_ref, y3_ref)
    for r in range(R):
        hs = h * rdeg[:, 2 * r:2 * r + 1]
        outs[r][...] = jnp.dot(hs, w2_ref[r],
                               preferred_element_type=jnp.float32)


def _t3_body(p_ref, rdeg_ref, b2_ref, out_ref):
    p = p_ref[...]
    rdeg = rdeg_ref[...]
    out = jnp.sum(b2_ref[...], axis=0)[None, :]
    for r in range(R):
        out = out + rdeg[:, 2 * r + 1:2 * r + 2] * (p[0, r] + p[1, r])
    out_ref[...] = out


_GRID = (N // BN,)

_t1_call = pl.pallas_call(
    _t1_body,
    grid=_GRID,
    in_specs=[
        pl.BlockSpec((BN, D), lambda i: (i, 0)),
        pl.BlockSpec((W, BN, 8), lambda i: (0, i, 0)),
        pl.BlockSpec((R, D, D), lambda i: (0, 0, 0)),
    ],
    out_specs=[
        pl.BlockSpec((BN, D), lambda i: (i, 0)),
        pl.BlockSpec((BN, D), lambda i: (i, 0)),
        pl.BlockSpec((BN, D), lambda i: (i, 0)),
        pl.BlockSpec((BN, D), lambda i: (i, 0)),
        pl.BlockSpec((BN, 8), lambda i: (i, 0)),
    ],
    out_shape=[
        jax.ShapeDtypeStruct((N, D), jnp.float32),
        jax.ShapeDtypeStruct((N, D), jnp.float32),
        jax.ShapeDtypeStruct((N, D), jnp.float32),
        jax.ShapeDtypeStruct((N, D), jnp.float32),
        jax.ShapeDtypeStruct((N, 8), jnp.float32),
    ],
)

_t2_call = pl.pallas_call(
    _t2_body,
    grid=_GRID,
    in_specs=[
        pl.BlockSpec((NC, R, BN, D), lambda i: (0, 0, i, 0)),
        pl.BlockSpec((BN, 8), lambda i: (i, 0)),
        pl.BlockSpec((R, D), lambda i: (0, 0)),
        pl.BlockSpec((R, D, D), lambda i: (0, 0, 0)),
    ],
    out_specs=[
        pl.BlockSpec((BN, D), lambda i: (i, 0)),
        pl.BlockSpec((BN, D), lambda i: (i, 0)),
        pl.BlockSpec((BN, D), lambda i: (i, 0)),
        pl.BlockSpec((BN, D), lambda i: (i, 0)),
    ],
    out_shape=[
        jax.ShapeDtypeStruct((N, D), jnp.float32),
        jax.ShapeDtypeStruct((N, D), jnp.float32),
        jax.ShapeDtypeStruct((N, D), jnp.float32),
        jax.ShapeDtypeStruct((N, D), jnp.float32),
    ],
)

_t3_call = pl.pallas_call(
    _t3_body,
    grid=_GRID,
    in_specs=[
        pl.BlockSpec((NC, R, BN, D), lambda i: (0, 0, i, 0)),
        pl.BlockSpec((BN, 8), lambda i: (i, 0)),
        pl.BlockSpec((R, D), lambda i: (0, 0)),
    ],
    out_specs=pl.BlockSpec((BN, D), lambda i: (i, 0)),
    out_shape=jax.ShapeDtypeStruct((N, D), jnp.float32),
)


def kernel(x, edge_index_0, edge_index_1, edge_index_2, edge_index_3,
           W1_0, W1_1, W1_2, W1_3, b1_0, b1_1, b1_2, b1_3,
           W2_0, W2_1, W2_2, W2_3, b2_0, b2_1, b2_2, b2_3):
    edges = (edge_index_0, edge_index_1, edge_index_2, edge_index_3)
    eflat = []
    for e in edges:
        e = e.astype(jnp.int32)
        eflat.append(e[0])
        eflat.append(e[1])

    cnt_parts = _hist_call(*eflat).reshape(W, N, 8)

    w1 = jnp.stack((W1_0, W1_1, W1_2, W1_3))
    b1 = jnp.stack((b1_0, b1_1, b1_2, b1_3))
    w2 = jnp.stack((W2_0, W2_1, W2_2, W2_3))
    b2 = jnp.stack((b2_0, b2_1, b2_2, b2_3))

    y10, y11, y12, y13, rdeg = _t1_call(x, cnt_parts, w1)
    p1 = _agg_call(y10, y11, y12, y13, *eflat).reshape(NC, R, NP, D)[:, :, :N, :]
    y20, y21, y22, y23 = _t2_call(p1, rdeg, b1, w2)
    p2 = _agg_call(y20, y21, y22, y23, *eflat).reshape(NC, R, NP, D)[:, :, :N, :]
    return _t3_call(p2, rdeg, b2)


# speedup vs baseline: 3.7157x; 1.5290x over previous
import jax
import jax.numpy as jnp
from jax import lax
from jax.experimental import pallas as pl
from jax.experimental.pallas import tpu as pltpu
from jax.experimental.pallas import tpu_sc as plsc

N = 10000
E = 80000
R = 4
D = 128

NC = 2
NS = 16
L = 16
W = NC * NS

K = 128
NCHUNKP = 640
CPT = NCHUNKP // W
EPAD = NCHUNKP * K
NP = 10240
RPT = NP // NS
ZR = 128
ZRZ = 32

BN = 1000


def _hist_body(s0, d0, s1, d1, s2, d2, s3, d3, out_hbm,
               hist, bufa, bufb, sema, semb):
    cid = lax.axis_index("c")
    sid = lax.axis_index("s")
    w = cid * NS + sid

    streams = (s0, d0, s1, d1, s2, d2, s3, d3)
    bufs = (bufa, bufb)
    sems = (sema, semb)

    pltpu.async_copy(streams[0].at[w], bufa, sema)

    zv = jnp.zeros((L,), jnp.float32)

    def zero_body(i, _):
        for u in range(8):
            hist[pl.ds(i * 128 + u * L, L)] = zv
        return 0

    lax.fori_loop(0, NP * 8 // 128, zero_body, 0)

    ones = jnp.ones((L,), jnp.float32)
    for s in range(8):
        buf = bufs[s % 2]
        sem = sems[s % 2]
        pltpu.make_async_copy(streams[s].at[w], buf, sem).wait()
        if s + 1 < 8:
            pltpu.async_copy(streams[s + 1].at[w], bufs[(s + 1) % 2],
                             sems[(s + 1) % 2])

        def body(g, _, buf=buf, col=s):
            j = g // 8
            o = g % 8
            idx = buf[j, pl.ds(o * L, L)]
            plsc.addupdate_scatter(hist, [idx * 8 + col], ones)
            return 0

        lax.fori_loop(0, CPT * 8, body, 0)

    pltpu.sync_copy(hist, out_hbm.at[w])


_hist_call = pl.kernel(
    _hist_body,
    out_type=jax.ShapeDtypeStruct((W, NP * 8), jnp.float32),
    mesh=plsc.VectorSubcoreMesh(core_axis_name="c", subcore_axis_name="s"),
    compiler_params=pltpu.CompilerParams(needs_layout_passes=False),
    scratch_types=[
        pltpu.VMEM((NP * 8,), jnp.float32),
        pltpu.VMEM((CPT, K), jnp.int32),
        pltpu.VMEM((CPT, K), jnp.int32),
        pltpu.SemaphoreType.DMA,
        pltpu.SemaphoreType.DMA,
    ],
)


def _agg_body(y0, y1, y2, y3, s0, d0, s1, d1, s2, d2, s3, d3, out_hbm,
              acc, rowsa, rowsb, zbuf, ssl, sdl,
              slabsem, ga, gb, sa, sb, zsem):
    cid = lax.axis_index("c")
    sid = lax.axis_index("s")
    w = cid * NS + sid

    ys = (y0, y1, y2, y3)
    es = (s0, s1, s2, s3)
    ed = (d0, d1, d2, d3)

    pltpu.async_copy(es[0].at[w], ssl, slabsem)
    pltpu.async_copy(ed[0].at[w], sdl, slabsem)

    zv = jnp.zeros((L,), jnp.float32)

    def zfill(i, _):
        for u in range(8):
            zbuf[i, pl.ds(u * L, L)] = zv
        return 0

    lax.fori_loop(0, ZRZ, zfill, 0)

    for j in range(RPT // ZRZ):
        pltpu.async_copy(zbuf, acc.at[pl.ds(sid * RPT + j * ZRZ, ZRZ)], zsem)
    for j in range(RPT // ZRZ):
        pltpu.make_async_copy(
            zbuf, acc.at[pl.ds(sid * RPT + j * ZRZ, ZRZ)], zsem).wait()

    pltpu.make_async_copy(es[0].at[w], ssl, slabsem).wait()
    pltpu.make_async_copy(ed[0].at[w], sdl, slabsem).wait()
    plsc.subcore_barrier()

    for r in range(R):
        y = ys[r]

        def body(t, _, y=y):
            ga_d = pltpu.async_copy(y.at[ssl.at[2 * t]], rowsa, ga)
            gb_d = pltpu.async_copy(y.at[ssl.at[2 * t + 1]], rowsb, gb)
            ga_d.wait()
            sa_d = pltpu.async_copy(rowsa, acc.at[sdl.at[2 * t]], sa,
                                    add=True)
            gb_d.wait()
            sb_d = pltpu.async_copy(rowsb, acc.at[sdl.at[2 * t + 1]], sb,
                                    add=True)
            sa_d.wait()
            sb_d.wait()
            return 0

        lax.fori_loop(0, CPT // 2, body, 0)
        if r + 1 < R:
            pltpu.async_copy(es[r + 1].at[w], ssl, slabsem)
            pltpu.async_copy(ed[r + 1].at[w], sdl, slabsem)
        plsc.subcore_barrier()

        slot = cid * R + r
        for j in range(RPT // ZR):
            ro = sid * RPT + j * ZR
            pltpu.async_copy(acc.at[pl.ds(ro, ZR)],
                             out_hbm.at[pl.ds(slot * NP + ro, ZR)], zsem)
        for j in range(RPT // ZR):
            ro = sid * RPT + j * ZR
            pltpu.make_async_copy(
                acc.at[pl.ds(ro, ZR)],
                out_hbm.at[pl.ds(slot * NP + ro, ZR)], zsem).wait()
        if r + 1 < R:
            for j in range(RPT // ZRZ):
                ro = sid * RPT + j * ZRZ
                pltpu.async_copy(zbuf, acc.at[pl.ds(ro, ZRZ)], zsem)
            for j in range(RPT // ZRZ):
                ro = sid * RPT + j * ZRZ
                pltpu.make_async_copy(zbuf, acc.at[pl.ds(ro, ZRZ)],
                                      zsem).wait()
            pltpu.make_async_copy(es[r + 1].at[w], ssl, slabsem).wait()
            pltpu.make_async_copy(ed[r + 1].at[w], sdl, slabsem).wait()
        plsc.subcore_barrier()


_agg_call = pl.kernel(
    _agg_body,
    out_type=jax.ShapeDtypeStruct((NC * R * NP, D), jnp.float32),
    mesh=plsc.VectorSubcoreMesh(core_axis_name="c", subcore_axis_name="s"),
    compiler_params=pltpu.CompilerParams(needs_layout_passes=False),
    scratch_types=[
        pltpu.VMEM_SHARED((NP, D), jnp.float32),
        pltpu.VMEM((K, D), jnp.float32),
        pltpu.VMEM((K, D), jnp.float32),
        pltpu.VMEM((ZRZ, D), jnp.float32),
        pltpu.VMEM((CPT, K), jnp.int32),
        pltpu.VMEM((CPT, K), jnp.int32),
        pltpu.SemaphoreType.DMA,
        pltpu.SemaphoreType.DMA,
        pltpu.SemaphoreType.DMA,
        pltpu.SemaphoreType.DMA,
        pltpu.SemaphoreType.DMA,
        pltpu.SemaphoreType.DMA,
    ],
)


def _t1_body(x_ref, cnt_ref, w1_ref, y0_ref, y1_ref, y2_ref, y3_ref,
             rdeg_ref):
    cnt = jnp.sum(cnt_ref[...], axis=0)
    rdeg = lax.rsqrt(jnp.maximum(cnt, 1.0))
    rdeg_ref[...] = rdeg
    x = x_ref[...]
    outs = (y0_ref, y1_ref, y2_ref, y3_ref)
    for r in range(R):
        xs = x * rdeg[:, 2 * r:2 * r + 1]
        outs[r][...] = jnp.dot(xs, w1_ref[r],
                               preferred_element_type=jnp.float32)


def _t2_body(p_ref, rdeg_ref, b1_ref, w2_ref, y0_ref, y1_ref, y2_ref,
             y3_ref):
    p = p_ref[...]
    rdeg = rdeg_ref[...]
    h = jnp.sum(b1_ref[...], axis=0)[None, :]
    for r in range(R):
        h = h + rdeg[:, 2 * r + 1:2 * r + 2] * (p[0, r] + p[1, r])
    h = jnp.maximum(h, 0.0)
    outs = (y0_ref, y1_ref, y2_ref, y3_ref)
    for r in range(R):
        hs = h * rdeg[:, 2 * r:2 * r + 1]
        outs[r][...] = jnp.dot(hs, w2_ref[r],
                               preferred_element_type=jnp.float32)


def _t3_body(p_ref, rdeg_ref, b2_ref, out_ref):
    p = p_ref[...]
    rdeg = rdeg_ref[...]
    out = jnp.sum(b2_ref[...], axis=0)[None, :]
    for r in range(R):
        out = out + rdeg[:, 2 * r + 1:2 * r + 2] * (p[0, r] + p[1, r])
    out_ref[...] = out


_GRID = (N // BN,)

_t1_call = pl.pallas_call(
    _t1_body,
    grid=_GRID,
    in_specs=[
        pl.BlockSpec((BN, D), lambda i: (i, 0)),
        pl.BlockSpec((W, BN, 8), lambda i: (0, i, 0)),
        pl.BlockSpec((R, D, D), lambda i: (0, 0, 0)),
    ],
    out_specs=[
        pl.BlockSpec((BN, D), lambda i: (i, 0)),
        pl.BlockSpec((BN, D), lambda i: (i, 0)),
        pl.BlockSpec((BN, D), lambda i: (i, 0)),
        pl.BlockSpec((BN, D), lambda i: (i, 0)),
        pl.BlockSpec((BN, 8), lambda i: (i, 0)),
    ],
    out_shape=[
        jax.ShapeDtypeStruct((NP, D), jnp.float32),
        jax.ShapeDtypeStruct((NP, D), jnp.float32),
        jax.ShapeDtypeStruct((NP, D), jnp.float32),
        jax.ShapeDtypeStruct((NP, D), jnp.float32),
        jax.ShapeDtypeStruct((N, 8), jnp.float32),
    ],
)

_t2_call = pl.pallas_call(
    _t2_body,
    grid=_GRID,
    in_specs=[
        pl.BlockSpec((NC, R, BN, D), lambda i: (0, 0, i, 0)),
        pl.BlockSpec((BN, 8), lambda i: (i, 0)),
        pl.BlockSpec((R, D), lambda i: (0, 0)),
        pl.BlockSpec((R, D, D), lambda i: (0, 0, 0)),
    ],
    out_specs=[
        pl.BlockSpec((BN, D), lambda i: (i, 0)),
        pl.BlockSpec((BN, D), lambda i: (i, 0)),
        pl.BlockSpec((BN, D), lambda i: (i, 0)),
        pl.BlockSpec((BN, D), lambda i: (i, 0)),
    ],
    out_shape=[
        jax.ShapeDtypeStruct((NP, D), jnp.float32),
        jax.ShapeDtypeStruct((NP, D), jnp.float32),
        jax.ShapeDtypeStruct((NP, D), jnp.float32),
        jax.ShapeDtypeStruct((NP, D), jnp.float32),
    ],
)

_t3_call = pl.pallas_call(
    _t3_body,
    grid=_GRID,
    in_specs=[
        pl.BlockSpec((NC, R, BN, D), lambda i: (0, 0, i, 0)),
        pl.BlockSpec((BN, 8), lambda i: (i, 0)),
        pl.BlockSpec((R, D), lambda i: (0, 0)),
    ],
    out_specs=pl.BlockSpec((BN, D), lambda i: (i, 0)),
    out_shape=jax.ShapeDtypeStruct((N, D), jnp.float32),
)


def kernel(x, edge_index_0, edge_index_1, edge_index_2, edge_index_3,
           W1_0, W1_1, W1_2, W1_3, b1_0, b1_1, b1_2, b1_3,
           W2_0, W2_1, W2_2, W2_3, b2_0, b2_1, b2_2, b2_3):
    edges = (edge_index_0, edge_index_1, edge_index_2, edge_index_3)
    pad = (N + (jnp.arange(EPAD - E, dtype=jnp.int32) % (NP - N)))
    eflat = []
    for e in edges:
        e = e.astype(jnp.int32)
        eflat.append(jnp.concatenate([e[0], pad]).reshape(W, CPT, K))
        eflat.append(jnp.concatenate([e[1], pad]).reshape(W, CPT, K))

    cnt_parts = _hist_call(*eflat).reshape(W, NP, 8)

    w1 = jnp.stack((W1_0, W1_1, W1_2, W1_3))
    b1 = jnp.stack((b1_0, b1_1, b1_2, b1_3))
    w2 = jnp.stack((W2_0, W2_1, W2_2, W2_3))
    b2 = jnp.stack((b2_0, b2_1, b2_2, b2_3))

    y10, y11, y12, y13, rdeg = _t1_call(x, cnt_parts, w1)
    p1 = _agg_call(y10, y11, y12, y13, *eflat).reshape(NC, R, NP, D)
    y20, y21, y22, y23 = _t2_call(p1, rdeg, b1, w2)
    p2 = _agg_call(y20, y21, y22, y23, *eflat).reshape(NC, R, NP, D)
    return _t3_call(p2, rdeg, b2)


# speedup vs baseline: 3.9437x; 1.0614x over previous
import jax
import jax.numpy as jnp
from jax import lax
from jax.experimental import pallas as pl
from jax.experimental.pallas import tpu as pltpu
from jax.experimental.pallas import tpu_sc as plsc

N = 10000
E = 80000
R = 4
D = 128

NC = 2
NS = 16
L = 16
W = NC * NS

K = 128
NCHUNKP = 640
CPT = NCHUNKP // W
EPAD = NCHUNKP * K
NP = 10240
RPT = NP // NS
ZR = 128
ZRZ = 32

BN = 1000


def _hist_body(e_all, out_hbm, hist, bufa, bufb, sema, semb):
    cid = lax.axis_index("c")
    sid = lax.axis_index("s")
    w = cid * NS + sid

    bufs = (bufa, bufb)
    sems = (sema, semb)

    pltpu.async_copy(e_all.at[0, w], bufa, sema)

    zv = jnp.zeros((L,), jnp.float32)

    def zero_body(i, _):
        for u in range(8):
            hist[pl.ds(i * 128 + u * L, L)] = zv
        return 0

    lax.fori_loop(0, NP * 8 // 128, zero_body, 0)

    ones = jnp.ones((L,), jnp.float32)
    for s in range(8):
        buf = bufs[s % 2]
        sem = sems[s % 2]
        pltpu.make_async_copy(e_all.at[s, w], buf, sem).wait()
        if s + 1 < 8:
            pltpu.async_copy(e_all.at[s + 1, w], bufs[(s + 1) % 2],
                             sems[(s + 1) % 2])

        def body(g, _, buf=buf, col=s):
            j = g // 8
            o = g % 8
            idx = buf[j, pl.ds(o * L, L)]
            plsc.addupdate_scatter(hist, [idx * 8 + col], ones)
            return 0

        lax.fori_loop(0, CPT * 8, body, 0)

    pltpu.sync_copy(hist, out_hbm.at[w])


_hist_call = pl.kernel(
    _hist_body,
    out_type=jax.ShapeDtypeStruct((W, NP * 8), jnp.float32),
    mesh=plsc.VectorSubcoreMesh(core_axis_name="c", subcore_axis_name="s"),
    compiler_params=pltpu.CompilerParams(needs_layout_passes=False),
    scratch_types=[
        pltpu.VMEM((NP * 8,), jnp.float32),
        pltpu.VMEM((CPT, K), jnp.int32),
        pltpu.VMEM((CPT, K), jnp.int32),
        pltpu.SemaphoreType.DMA,
        pltpu.SemaphoreType.DMA,
    ],
)


def _agg_body(y0, y1, y2, y3, e_all, out_hbm,
              acc, rowsa, rowsb, zbuf, ssl, sdl,
              slabsem, ga, gb, sa, sb, zsem):
    cid = lax.axis_index("c")
    sid = lax.axis_index("s")
    w = cid * NS + sid

    ys = (y0, y1, y2, y3)

    pltpu.async_copy(e_all.at[0, w], ssl, slabsem)
    pltpu.async_copy(e_all.at[1, w], sdl, slabsem)

    zv = jnp.zeros((L,), jnp.float32)

    def zfill(i, _):
        for u in range(8):
            zbuf[i, pl.ds(u * L, L)] = zv
        return 0

    lax.fori_loop(0, ZRZ, zfill, 0)

    for j in range(RPT // ZRZ):
        pltpu.async_copy(zbuf, acc.at[pl.ds(sid * RPT + j * ZRZ, ZRZ)], zsem)
    for j in range(RPT // ZRZ):
        pltpu.make_async_copy(
            zbuf, acc.at[pl.ds(sid * RPT + j * ZRZ, ZRZ)], zsem).wait()

    pltpu.make_async_copy(e_all.at[0, w], ssl, slabsem).wait()
    pltpu.make_async_copy(e_all.at[1, w], sdl, slabsem).wait()
    plsc.subcore_barrier()

    g0 = pltpu.async_copy(y0.at[ssl.at[0]], rowsa, ga)
    g1 = pltpu.async_copy(y0.at[ssl.at[1]], rowsb, gb)
    for r in range(R):
        y = ys[r]
        g0.wait()
        s0 = pltpu.async_copy(rowsa, acc.at[sdl.at[0]], sa, add=True)
        g1.wait()
        s1 = pltpu.async_copy(rowsb, acc.at[sdl.at[1]], sb, add=True)
        s0.wait()
        s1.wait()

        def body(t, _, y=y):
            ga_d = pltpu.async_copy(y.at[ssl.at[2 * t]], rowsa, ga)
            gb_d = pltpu.async_copy(y.at[ssl.at[2 * t + 1]], rowsb, gb)
            ga_d.wait()
            sa_d = pltpu.async_copy(rowsa, acc.at[sdl.at[2 * t]], sa,
                                    add=True)
            gb_d.wait()
            sb_d = pltpu.async_copy(rowsb, acc.at[sdl.at[2 * t + 1]], sb,
                                    add=True)
            sa_d.wait()
            sb_d.wait()
            return 0

        lax.fori_loop(1, CPT // 2, body, 0)
        if r + 1 < R:
            pltpu.async_copy(e_all.at[2 * r + 2, w], ssl, slabsem)
            pltpu.async_copy(e_all.at[2 * r + 3, w], sdl, slabsem)
        plsc.subcore_barrier()

        slot = cid * R + r
        exps = []
        for j in range(RPT // ZR):
            ro = sid * RPT + j * ZR
            exps.append(pltpu.async_copy(
                acc.at[pl.ds(ro, ZR)],
                out_hbm.at[pl.ds(slot * NP + ro, ZR)], zsem))
        if r + 1 < R:
            pltpu.make_async_copy(e_all.at[2 * r + 2, w], ssl,
                                  slabsem).wait()
            pltpu.make_async_copy(e_all.at[2 * r + 3, w], sdl,
                                  slabsem).wait()
            g0 = pltpu.async_copy(ys[r + 1].at[ssl.at[0]], rowsa, ga)
            g1 = pltpu.async_copy(ys[r + 1].at[ssl.at[1]], rowsb, gb)
        for e in exps:
            e.wait()
        plsc.subcore_barrier()


_agg_call = pl.kernel(
    _agg_body,
    out_type=jax.ShapeDtypeStruct((NC * R * NP, D), jnp.float32),
    mesh=plsc.VectorSubcoreMesh(core_axis_name="c", subcore_axis_name="s"),
    compiler_params=pltpu.CompilerParams(needs_layout_passes=False),
    scratch_types=[
        pltpu.VMEM_SHARED((NP, D), jnp.float32),
        pltpu.VMEM((K, D), jnp.float32),
        pltpu.VMEM((K, D), jnp.float32),
        pltpu.VMEM((ZRZ, D), jnp.float32),
        pltpu.VMEM((CPT, K), jnp.int32),
        pltpu.VMEM((CPT, K), jnp.int32),
        pltpu.SemaphoreType.DMA,
        pltpu.SemaphoreType.DMA,
        pltpu.SemaphoreType.DMA,
        pltpu.SemaphoreType.DMA,
        pltpu.SemaphoreType.DMA,
        pltpu.SemaphoreType.DMA,
    ],
)


def _t1_body(x_ref, cnt_ref, w1_ref, y0_ref, y1_ref, y2_ref, y3_ref,
             rdeg_ref):
    cnt = jnp.sum(cnt_ref[...], axis=0)
    rdeg = lax.rsqrt(jnp.maximum(cnt, 1.0))
    rdeg_ref[...] = rdeg
    x = x_ref[...]
    outs = (y0_ref, y1_ref, y2_ref, y3_ref)
    for r in range(R):
        xs = x * rdeg[:, 2 * r:2 * r + 1]
        outs[r][...] = jnp.dot(xs, w1_ref[r],
                               preferred_element_type=jnp.float32)


def _t2_body(p_ref, rdeg_ref, b1_ref, w2_ref, y0_ref, y1_ref, y2_ref,
             y3_ref):
    p = p_ref[...]
    rdeg = rdeg_ref[...]
    c = [p[0, r] + p[1, r] for r in range(R)]
    h = jnp.sum(b1_ref[...], axis=0)[None, :]
    for r in range(R):
        pr = c[r] - c[r - 1] if r > 0 else c[r]
        h = h + rdeg[:, 2 * r + 1:2 * r + 2] * pr
    h = jnp.maximum(h, 0.0)
    outs = (y0_ref, y1_ref, y2_ref, y3_ref)
    for r in range(R):
        hs = h * rdeg[:, 2 * r:2 * r + 1]
        outs[r][...] = jnp.dot(hs, w2_ref[r],
                               preferred_element_type=jnp.float32)


def _t3_body(p_ref, rdeg_ref, b2_ref, out_ref):
    p = p_ref[...]
    rdeg = rdeg_ref[...]
    c = [p[0, r] + p[1, r] for r in range(R)]
    out = jnp.sum(b2_ref[...], axis=0)[None, :]
    for r in range(R):
        pr = c[r] - c[r - 1] if r > 0 else c[r]
        out = out + rdeg[:, 2 * r + 1:2 * r + 2] * pr
    out_ref[...] = out


_GRID = (N // BN,)

_t1_call = pl.pallas_call(
    _t1_body,
    grid=_GRID,
    in_specs=[
        pl.BlockSpec((BN, D), lambda i: (i, 0)),
        pl.BlockSpec((W, BN, 8), lambda i: (0, i, 0)),
        pl.BlockSpec((R, D, D), lambda i: (0, 0, 0)),
    ],
    out_specs=[
        pl.BlockSpec((BN, D), lambda i: (i, 0)),
        pl.BlockSpec((BN, D), lambda i: (i, 0)),
        pl.BlockSpec((BN, D), lambda i: (i, 0)),
        pl.BlockSpec((BN, D), lambda i: (i, 0)),
        pl.BlockSpec((BN, 8), lambda i: (i, 0)),
    ],
    out_shape=[
        jax.ShapeDtypeStruct((NP, D), jnp.float32),
        jax.ShapeDtypeStruct((NP, D), jnp.float32),
        jax.ShapeDtypeStruct((NP, D), jnp.float32),
        jax.ShapeDtypeStruct((NP, D), jnp.float32),
        jax.ShapeDtypeStruct((N, 8), jnp.float32),
    ],
)

_t2_call = pl.pallas_call(
    _t2_body,
    grid=_GRID,
    in_specs=[
        pl.BlockSpec((NC, R, BN, D), lambda i: (0, 0, i, 0)),
        pl.BlockSpec((BN, 8), lambda i: (i, 0)),
        pl.BlockSpec((R, D), lambda i: (0, 0)),
        pl.BlockSpec((R, D, D), lambda i: (0, 0, 0)),
    ],
    out_specs=[
        pl.BlockSpec((BN, D), lambda i: (i, 0)),
        pl.BlockSpec((BN, D), lambda i: (i, 0)),
        pl.BlockSpec((BN, D), lambda i: (i, 0)),
        pl.BlockSpec((BN, D), lambda i: (i, 0)),
    ],
    out_shape=[
        jax.ShapeDtypeStruct((NP, D), jnp.float32),
        jax.ShapeDtypeStruct((NP, D), jnp.float32),
        jax.ShapeDtypeStruct((NP, D), jnp.float32),
        jax.ShapeDtypeStruct((NP, D), jnp.float32),
    ],
)

_t3_call = pl.pallas_call(
    _t3_body,
    grid=_GRID,
    in_specs=[
        pl.BlockSpec((NC, R, BN, D), lambda i: (0, 0, i, 0)),
        pl.BlockSpec((BN, 8), lambda i: (i, 0)),
        pl.BlockSpec((R, D), lambda i: (0, 0)),
    ],
    out_specs=pl.BlockSpec((BN, D), lambda i: (i, 0)),
    out_shape=jax.ShapeDtypeStruct((N, D), jnp.float32),
)


def kernel(x, edge_index_0, edge_index_1, edge_index_2, edge_index_3,
           W1_0, W1_1, W1_2, W1_3, b1_0, b1_1, b1_2, b1_3,
           W2_0, W2_1, W2_2, W2_3, b2_0, b2_1, b2_2, b2_3):
    edges = (edge_index_0, edge_index_1, edge_index_2, edge_index_3)
    pad = (N + (jnp.arange(EPAD - E, dtype=jnp.int32) % (NP - N)))
    parts = []
    for e in edges:
        e = e.astype(jnp.int32)
        parts += [e[0], pad, e[1], pad]
    e_all = jnp.concatenate(parts).reshape(2 * R, W, CPT, K)

    cnt_parts = _hist_call(e_all).reshape(W, NP, 8)

    w1 = jnp.stack((W1_0, W1_1, W1_2, W1_3))
    b1 = jnp.stack((b1_0, b1_1, b1_2, b1_3))
    w2 = jnp.stack((W2_0, W2_1, W2_2, W2_3))
    b2 = jnp.stack((b2_0, b2_1, b2_2, b2_3))

    y10, y11, y12, y13, rdeg = _t1_call(x, cnt_parts, w1)
    p1 = _agg_call(y10, y11, y12, y13, e_all).reshape(NC, R, NP, D)
    y20, y21, y22, y23 = _t2_call(p1, rdeg, b1, w2)
    p2 = _agg_call(y20, y21, y22, y23, e_all).reshape(NC, R, NP, D)
    return _t3_call(p2, rdeg, b2)


# speedup vs baseline: 3.9541x; 1.0026x over previous
import jax
import jax.numpy as jnp
from jax import lax
from jax.experimental import pallas as pl
from jax.experimental.pallas import tpu as pltpu
from jax.experimental.pallas import tpu_sc as plsc

N = 10000
E = 80000
R = 4
D = 128

NC = 2
NS = 16
L = 16
W = NC * NS

K = 128
NCHUNKP = 640
CPT = NCHUNKP // W
EPAD = NCHUNKP * K
NP = 10240
RPT = NP // NS
ZR = 128
ZRZ = 32

BN = 1000


def _hist_body(e_all, out_hbm, hist, bufa, bufb, sema, semb):
    cid = lax.axis_index("c")
    sid = lax.axis_index("s")
    w = cid * NS + sid

    bufs = (bufa, bufb)
    sems = (sema, semb)

    pltpu.async_copy(e_all.at[0, w], bufa, sema)

    zv = jnp.zeros((L,), jnp.float32)

    def zero_body(i, _):
        for u in range(8):
            hist[pl.ds(i * 128 + u * L, L)] = zv
        return 0

    lax.fori_loop(0, NP * 8 // 128, zero_body, 0)

    ones = jnp.ones((L,), jnp.float32)
    for s in range(8):
        buf = bufs[s % 2]
        sem = sems[s % 2]
        pltpu.make_async_copy(e_all.at[s, w], buf, sem).wait()
        if s + 1 < 8:
            pltpu.async_copy(e_all.at[s + 1, w], bufs[(s + 1) % 2],
                             sems[(s + 1) % 2])

        def body(g, _, buf=buf, col=s):
            j = g // 8
            o = g % 8
            idx = buf[j, pl.ds(o * L, L)]
            plsc.addupdate_scatter(hist, [idx * 8 + col], ones)
            return 0

        lax.fori_loop(0, CPT * 8, body, 0)

    pltpu.sync_copy(hist, out_hbm.at[w])


_hist_call = pl.kernel(
    _hist_body,
    out_type=jax.ShapeDtypeStruct((W, NP * 8), jnp.float32),
    mesh=plsc.VectorSubcoreMesh(core_axis_name="c", subcore_axis_name="s"),
    compiler_params=pltpu.CompilerParams(needs_layout_passes=False),
    scratch_types=[
        pltpu.VMEM((NP * 8,), jnp.float32),
        pltpu.VMEM((CPT, K), jnp.int32),
        pltpu.VMEM((CPT, K), jnp.int32),
        pltpu.SemaphoreType.DMA,
        pltpu.SemaphoreType.DMA,
    ],
)


def _agg_body(y0, y1, y2, y3, e_all, out_hbm,
              acc, rowsa, rowsb, zbuf, ssl, sdl,
              slabsem, ga, gb, sa, sb, zsem):
    cid = lax.axis_index("c")
    sid = lax.axis_index("s")
    w = cid * NS + sid

    ys = (y0, y1, y2, y3)

    pltpu.async_copy(e_all.at[0, w], ssl, slabsem)
    pltpu.async_copy(e_all.at[1, w], sdl, slabsem)

    zv = jnp.zeros((L,), jnp.float32)

    def zfill(i, _):
        for u in range(8):
            zbuf[i, pl.ds(u * L, L)] = zv
        return 0

    lax.fori_loop(0, ZRZ, zfill, 0)

    for j in range(RPT // ZRZ):
        pltpu.async_copy(zbuf, acc.at[pl.ds(sid * RPT + j * ZRZ, ZRZ)], zsem)
    for j in range(RPT // ZRZ):
        pltpu.make_async_copy(
            zbuf, acc.at[pl.ds(sid * RPT + j * ZRZ, ZRZ)], zsem).wait()

    pltpu.make_async_copy(e_all.at[0, w], ssl, slabsem).wait()
    pltpu.make_async_copy(e_all.at[1, w], sdl, slabsem).wait()
    plsc.subcore_barrier()

    g0 = pltpu.async_copy(y0.at[ssl.at[0]], rowsa, ga)
    g1 = pltpu.async_copy(y0.at[ssl.at[1]], rowsb, gb)
    for r in range(R):
        y = ys[r]
        g0.wait()
        s0 = pltpu.async_copy(rowsa, acc.at[sdl.at[0]], sa, add=True)
        g1.wait()
        s1 = pltpu.async_copy(rowsb, acc.at[sdl.at[1]], sb, add=True)
        s0.wait()
        s1.wait()

        def body(t, _, y=y):
            ga_d = pltpu.async_copy(y.at[ssl.at[2 * t]], rowsa, ga)
            gb_d = pltpu.async_copy(y.at[ssl.at[2 * t + 1]], rowsb, gb)
            ga_d.wait()
            sa_d = pltpu.async_copy(rowsa, acc.at[sdl.at[2 * t]], sa,
                                    add=True)
            gb_d.wait()
            sb_d = pltpu.async_copy(rowsb, acc.at[sdl.at[2 * t + 1]], sb,
                                    add=True)
            sa_d.wait()
            sb_d.wait()
            return 0

        lax.fori_loop(1, CPT // 2, body, 0)
        if r + 1 < R:
            pltpu.async_copy(e_all.at[2 * r + 2, w], ssl, slabsem)
            pltpu.async_copy(e_all.at[2 * r + 3, w], sdl, slabsem)
        plsc.subcore_barrier()

        slot = cid * R + r
        exps = []
        for j in range(RPT // ZR):
            ro = sid * RPT + j * ZR
            exps.append(pltpu.async_copy(
                acc.at[pl.ds(ro, ZR)],
                out_hbm.at[pl.ds(slot * NP + ro, ZR)], zsem))
        if r + 1 < R:
            pltpu.make_async_copy(e_all.at[2 * r + 2, w], ssl,
                                  slabsem).wait()
            pltpu.make_async_copy(e_all.at[2 * r + 3, w], sdl,
                                  slabsem).wait()
            g0 = pltpu.async_copy(ys[r + 1].at[ssl.at[0]], rowsa, ga)
            g1 = pltpu.async_copy(ys[r + 1].at[ssl.at[1]], rowsb, gb)
        for e in exps:
            e.wait()
        plsc.subcore_barrier()


_agg_call = pl.kernel(
    _agg_body,
    out_type=jax.ShapeDtypeStruct((NC * R * NP, D), jnp.float32),
    mesh=plsc.VectorSubcoreMesh(core_axis_name="c", subcore_axis_name="s"),
    compiler_params=pltpu.CompilerParams(needs_layout_passes=False),
    scratch_types=[
        pltpu.VMEM_SHARED((NP, D), jnp.float32),
        pltpu.VMEM((K, D), jnp.float32),
        pltpu.VMEM((K, D), jnp.float32),
        pltpu.VMEM((ZRZ, D), jnp.float32),
        pltpu.VMEM((CPT, K), jnp.int32),
        pltpu.VMEM((CPT, K), jnp.int32),
        pltpu.SemaphoreType.DMA,
        pltpu.SemaphoreType.DMA,
        pltpu.SemaphoreType.DMA,
        pltpu.SemaphoreType.DMA,
        pltpu.SemaphoreType.DMA,
        pltpu.SemaphoreType.DMA,
    ],
)


def _t1_body(x_ref, cnt_ref, w10_ref, w11_ref, w12_ref, w13_ref,
             y0_ref, y1_ref, y2_ref, y3_ref, rdeg_ref):
    cnt = jnp.sum(cnt_ref[...], axis=0)
    rdeg = lax.rsqrt(jnp.maximum(cnt, 1.0))
    rdeg_ref[...] = rdeg
    x = x_ref[...]
    outs = (y0_ref, y1_ref, y2_ref, y3_ref)
    ws = (w10_ref, w11_ref, w12_ref, w13_ref)
    for r in range(R):
        xs = x * rdeg[:, 2 * r:2 * r + 1]
        outs[r][...] = jnp.dot(xs, ws[r][...],
                               preferred_element_type=jnp.float32)


def _t2_body(p_ref, rdeg_ref, b10_ref, b11_ref, b12_ref, b13_ref,
             w20_ref, w21_ref, w22_ref, w23_ref,
             y0_ref, y1_ref, y2_ref, y3_ref):
    p = p_ref[...]
    rdeg = rdeg_ref[...]
    c = [p[0, r] + p[1, r] for r in range(R)]
    h = (b10_ref[...] + b11_ref[...] + b12_ref[...] + b13_ref[...])[None, :]
    for r in range(R):
        pr = c[r] - c[r - 1] if r > 0 else c[r]
        h = h + rdeg[:, 2 * r + 1:2 * r + 2] * pr
    h = jnp.maximum(h, 0.0)
    outs = (y0_ref, y1_ref, y2_ref, y3_ref)
    ws = (w20_ref, w21_ref, w22_ref, w23_ref)
    for r in range(R):
        hs = h * rdeg[:, 2 * r:2 * r + 1]
        outs[r][...] = jnp.dot(hs, ws[r][...],
                               preferred_element_type=jnp.float32)


def _t3_body(p_ref, rdeg_ref, b20_ref, b21_ref, b22_ref, b23_ref,
             out_ref):
    p = p_ref[...]
    rdeg = rdeg_ref[...]
    c = [p[0, r] + p[1, r] for r in range(R)]
    out = (b20_ref[...] + b21_ref[...] + b22_ref[...] + b23_ref[...])[None, :]
    for r in range(R):
        pr = c[r] - c[r - 1] if r > 0 else c[r]
        out = out + rdeg[:, 2 * r + 1:2 * r + 2] * pr
    out_ref[...] = out


_GRID = (N // BN,)

_t1_call = pl.pallas_call(
    _t1_body,
    grid=_GRID,
    in_specs=[
        pl.BlockSpec((BN, D), lambda i: (i, 0)),
        pl.BlockSpec((W, BN, 8), lambda i: (0, i, 0)),
        pl.BlockSpec((D, D), lambda i: (0, 0)),
        pl.BlockSpec((D, D), lambda i: (0, 0)),
        pl.BlockSpec((D, D), lambda i: (0, 0)),
        pl.BlockSpec((D, D), lambda i: (0, 0)),
    ],
    out_specs=[
        pl.BlockSpec((BN, D), lambda i: (i, 0)),
        pl.BlockSpec((BN, D), lambda i: (i, 0)),
        pl.BlockSpec((BN, D), lambda i: (i, 0)),
        pl.BlockSpec((BN, D), lambda i: (i, 0)),
        pl.BlockSpec((BN, 8), lambda i: (i, 0)),
    ],
    out_shape=[
        jax.ShapeDtypeStruct((NP, D), jnp.float32),
        jax.ShapeDtypeStruct((NP, D), jnp.float32),
        jax.ShapeDtypeStruct((NP, D), jnp.float32),
        jax.ShapeDtypeStruct((NP, D), jnp.float32),
        jax.ShapeDtypeStruct((N, 8), jnp.float32),
    ],
)

_t2_call = pl.pallas_call(
    _t2_body,
    grid=_GRID,
    in_specs=[
        pl.BlockSpec((NC, R, BN, D), lambda i: (0, 0, i, 0)),
        pl.BlockSpec((BN, 8), lambda i: (i, 0)),
        pl.BlockSpec((D,), lambda i: (0,)),
        pl.BlockSpec((D,), lambda i: (0,)),
        pl.BlockSpec((D,), lambda i: (0,)),
        pl.BlockSpec((D,), lambda i: (0,)),
        pl.BlockSpec((D, D), lambda i: (0, 0)),
        pl.BlockSpec((D, D), lambda i: (0, 0)),
        pl.BlockSpec((D, D), lambda i: (0, 0)),
        pl.BlockSpec((D, D), lambda i: (0, 0)),
    ],
    out_specs=[
        pl.BlockSpec((BN, D), lambda i: (i, 0)),
        pl.BlockSpec((BN, D), lambda i: (i, 0)),
        pl.BlockSpec((BN, D), lambda i: (i, 0)),
        pl.BlockSpec((BN, D), lambda i: (i, 0)),
    ],
    out_shape=[
        jax.ShapeDtypeStruct((NP, D), jnp.float32),
        jax.ShapeDtypeStruct((NP, D), jnp.float32),
        jax.ShapeDtypeStruct((NP, D), jnp.float32),
        jax.ShapeDtypeStruct((NP, D), jnp.float32),
    ],
)

_t3_call = pl.pallas_call(
    _t3_body,
    grid=_GRID,
    in_specs=[
        pl.BlockSpec((NC, R, BN, D), lambda i: (0, 0, i, 0)),
        pl.BlockSpec((BN, 8), lambda i: (i, 0)),
        pl.BlockSpec((D,), lambda i: (0,)),
        pl.BlockSpec((D,), lambda i: (0,)),
        pl.BlockSpec((D,), lambda i: (0,)),
        pl.BlockSpec((D,), lambda i: (0,)),
    ],
    out_specs=pl.BlockSpec((BN, D), lambda i: (i, 0)),
    out_shape=jax.ShapeDtypeStruct((N, D), jnp.float32),
)


def kernel(x, edge_index_0, edge_index_1, edge_index_2, edge_index_3,
           W1_0, W1_1, W1_2, W1_3, b1_0, b1_1, b1_2, b1_3,
           W2_0, W2_1, W2_2, W2_3, b2_0, b2_1, b2_2, b2_3):
    edges = (edge_index_0, edge_index_1, edge_index_2, edge_index_3)
    pad = (N + (jnp.arange(EPAD - E, dtype=jnp.int32) % (NP - N)))
    parts = []
    for e in edges:
        e = e.astype(jnp.int32)
        parts += [e[0], pad, e[1], pad]
    e_all = jnp.concatenate(parts).reshape(2 * R, W, CPT, K)

    cnt_parts = _hist_call(e_all).reshape(W, NP, 8)

    y10, y11, y12, y13, rdeg = _t1_call(x, cnt_parts, W1_0, W1_1, W1_2,
                                        W1_3)
    p1 = _agg_call(y10, y11, y12, y13, e_all).reshape(NC, R, NP, D)
    y20, y21, y22, y23 = _t2_call(p1, rdeg, b1_0, b1_1, b1_2, b1_3,
                                  W2_0, W2_1, W2_2, W2_3)
    p2 = _agg_call(y20, y21, y22, y23, e_all).reshape(NC, R, NP, D)
    return _t3_call(p2, rdeg, b2_0, b2_1, b2_2, b2_3)
